# Initial kernel scaffold; baseline (speedup 1.0000x reference)
import jax
import jax.numpy as jnp
from jax.experimental import pallas as pl


def kernel(nodes_feat, edge_index, edges_feat, nodes_fc, edges_fc, edge_emb, nodes_attn, edges_attn, W, b, gamma, beta):
    raise NotImplementedError("write your pallas kernel here")


# speedup vs baseline: 18.1358x; 18.1358x over previous
import dataclasses
import functools

import jax
import jax.numpy as jnp
from jax import lax
from jax.experimental import pallas as pl
from jax.experimental.pallas import tpu as pltpu
from jax.experimental.pallas import tpu_sc as plsc

N = 10000
E = 320000
D = 128

NC = 2
NS = 16
NW = NC * NS
EPW = E // NW
CB = 80
NCHUNK = EPW // CB
ZCH = N // CB


def _pre_body(nf_ref, nfc_ref, nattn_ref, eemb_ref, efc_ref, eattn_ref,
              h_ref, ft_ref, ee_ref):
    hv = nf_ref[...] * nfc_ref[0:1, :]
    h_ref[...] = hv
    ft_ref[...] = jnp.sum(hv * nattn_ref[...], axis=1, keepdims=True)
    ee = jnp.sum(eemb_ref[...] * efc_ref[...] * eattn_ref[...], axis=1,
                 keepdims=True)
    ee_ref[...] = jnp.concatenate([ee, jnp.zeros((8, 1), jnp.float32)], axis=0)


def _edge_body(h_hbm, ft_hbm, ee_hbm, src_hbm, dst_hbm, et_hbm, out_hbm,
               ft_v, ee_v, src_v, et_v, dstt, a_v, rows_v, agg_sh, sem):
    cid = lax.axis_index("c")
    sid = lax.axis_index("s")
    wid = cid * NS + sid
    base0 = wid * EPW

    pltpu.sync_copy(ft_hbm, ft_v)
    pltpu.sync_copy(ee_hbm, ee_v)
    pltpu.sync_copy(src_hbm.at[pl.ds(base0, EPW)], src_v)
    pltpu.sync_copy(et_hbm.at[pl.ds(base0, EPW)], et_v)

    @pl.loop(0, CB)
    def _(r):
        for j in range(D // 16):
            rows_v[r, pl.ds(j * 16, 16)] = jnp.zeros((16,), jnp.float32)

    @pl.loop(0, ZCH)
    def _(k):
        @pl.when(lax.rem(k, NS) == sid)
        def _():
            pltpu.sync_copy(rows_v, agg_sh.at[pl.ds(k * CB, CB)])

    plsc.subcore_barrier()

    @pl.loop(0, NCHUNK)
    def _(c):
        gather = pltpu.async_copy(
            h_hbm.at[src_v.at[pl.ds(c * CB, CB)]], rows_v, sem)
        pltpu.sync_copy(dst_hbm.at[pl.ds(base0 + c * CB, CB)], dstt.at[0])

        @pl.loop(0, CB, step=16)
        def _(i):
            s16 = src_v[pl.ds(c * CB + i, 16)]
            d16 = dstt[0, pl.ds(i, 16)]
            t16 = et_v[pl.ds(c * CB + i, 16)]
            x = (plsc.load_gather(ft_v, [s16])
                 + plsc.load_gather(ft_v, [d16])
                 + plsc.load_gather(ee_v, [t16]))
            a_v[pl.ds(i, 16)] = jnp.maximum(x, x * 0.02)

        gather.wait()

        @pl.loop(0, CB)
        def _(e):
            ak = plsc.load_gather(a_v, [jnp.full((16,), e, jnp.int32)])
            for j in range(D // 16):
                sl = pl.ds(j * 16, 16)
                rows_v[e, sl] = rows_v[e, sl] * ak

        pltpu.sync_copy(rows_v, agg_sh.at[dstt.at[0]], add=True)

    plsc.subcore_barrier()

    @pl.loop(0, ZCH)
    def _(k):
        @pl.when(lax.rem(k, NS) == sid)
        def _():
            pltpu.sync_copy(agg_sh.at[pl.ds(k * CB, CB)],
                            out_hbm.at[cid, pl.ds(k * CB, CB)])


def _post_body(aggp_ref, w_ref, b_ref, gamma_ref, beta_ref, out_ref):
    agg = aggp_ref[0] + aggp_ref[1]
    y = lax.dot_general(agg, w_ref[...], (((1,), (1,)), ((), ())),
                        preferred_element_type=jnp.float32)
    y = y + b_ref[...]
    mean = jnp.mean(y, axis=0, keepdims=True)
    var = jnp.mean((y - mean) ** 2, axis=0, keepdims=True)
    yn = (y - mean) / jnp.sqrt(var + 1e-5) * gamma_ref[...] + beta_ref[...]
    out_ref[...] = jnp.where(yn > 0, yn, jnp.exp(jnp.minimum(yn, 0.0)) - 1.0)


def kernel(nodes_feat, edge_index, edges_feat, nodes_fc, edges_fc, edge_emb,
           nodes_attn, edges_attn, W, b, gamma, beta):
    h, ft2, ee2 = pl.pallas_call(
        _pre_body,
        out_shape=(
            jax.ShapeDtypeStruct((N, D), jnp.float32),
            jax.ShapeDtypeStruct((N, 1), jnp.float32),
            jax.ShapeDtypeStruct((16, 1), jnp.float32),
        ),
    )(nodes_feat, nodes_fc, nodes_attn, edge_emb, edges_fc, edges_attn)
    ft = ft2.reshape(N)
    ee = ee2.reshape(16)

    src = edge_index[0]
    dst = edge_index[1]

    cp = pltpu.CompilerParams()
    if "needs_layout_passes" in pltpu.CompilerParams.__dataclass_fields__:
        cp = dataclasses.replace(cp, needs_layout_passes=False)
    mesh = plsc.VectorSubcoreMesh(core_axis_name="c", subcore_axis_name="s")
    edge_kernel = pl.kernel(
        _edge_body,
        compiler_params=cp,
        out_type=jax.ShapeDtypeStruct((NC, N, D), jnp.float32),
        mesh=mesh,
        scratch_types=[
            pltpu.VMEM((N,), jnp.float32),
            pltpu.VMEM((16,), jnp.float32),
            pltpu.VMEM((EPW,), jnp.int32),
            pltpu.VMEM((EPW,), jnp.int32),
            pltpu.VMEM((1, CB), jnp.int32),
            pltpu.VMEM((CB,), jnp.float32),
            pltpu.VMEM((CB, D), jnp.float32),
            pltpu.VMEM_SHARED((N, D), jnp.float32),
            pltpu.SemaphoreType.DMA,
        ],
    )
    aggp = edge_kernel(h, ft, ee, src, dst, edges_feat)

    out = pl.pallas_call(
        _post_body,
        out_shape=jax.ShapeDtypeStruct((N, D), jnp.float32),
    )(aggp, W, b.reshape(1, D), gamma.reshape(1, D), beta.reshape(1, D))
    return out


# speedup vs baseline: 27.1662x; 1.4979x over previous
import dataclasses
import functools

import jax
import jax.numpy as jnp
from jax import lax
from jax.experimental import pallas as pl
from jax.experimental.pallas import tpu as pltpu
from jax.experimental.pallas import tpu_sc as plsc

N = 10000
E = 320000
D = 128

NC = 2
NS = 16
NW = NC * NS
EPW = E // NW
CB = 80
NCHUNK = EPW // CB
ZCH = N // CB


def _pre_body(nf_ref, nfc_ref, nattn_ref, eemb_ref, efc_ref, eattn_ref,
              h_ref, ft_ref, ee_ref):
    hv = nf_ref[...] * nfc_ref[0:1, :]
    h_ref[...] = hv
    ft_ref[...] = jnp.sum(hv * nattn_ref[...], axis=1, keepdims=True)
    ee = jnp.sum(eemb_ref[...] * efc_ref[...] * eattn_ref[...], axis=1,
                 keepdims=True)
    ee_ref[...] = jnp.concatenate([ee, jnp.zeros((8, 1), jnp.float32)], axis=0)


def _edge_body(h_hbm, ft_hbm, ee_hbm, pk_hbm, dst_hbm, out_hbm,
               ft_v, ee_v, pk_v, dstc, a_v, sc0, sc1, rows0, rows1,
               agg_sh, semg0, semg1, semd0, semd1):
    cid = lax.axis_index("c")
    sid = lax.axis_index("s")
    wid = cid * NS + sid
    base0 = wid * EPW
    rows = (rows0, rows1)
    sc = (sc0, sc1)
    semg = (semg0, semg1)
    semd = (semd0, semd1)

    pltpu.sync_copy(ft_hbm, ft_v)
    pltpu.sync_copy(ee_hbm, ee_v)
    pltpu.sync_copy(pk_hbm.at[pl.ds(base0, EPW)], pk_v)

    @pl.loop(0, CB)
    def _(r):
        for j in range(D // 16):
            rows0[r, pl.ds(j * 16, 16)] = jnp.zeros((16,), jnp.float32)

    @pl.loop(0, ZCH)
    def _(k):
        @pl.when(lax.rem(k, NS) == sid)
        def _():
            pltpu.sync_copy(rows0, agg_sh.at[pl.ds(k * CB, CB)])

    plsc.subcore_barrier()

    def prefetch(k, c):
        pltpu.async_copy(dst_hbm.at[pl.ds(base0 + c * CB, CB)],
                         dstc.at[k], semd[k])

        @pl.loop(0, CB, step=16)
        def _(i):
            sc[k][pl.ds(i, 16)] = pk_v[pl.ds(c * CB + i, 16)] & 0x3FFF

        pltpu.async_copy(h_hbm.at[sc[k]], rows[k], semg[k])

    def process(k, c):
        pltpu.make_async_copy(dst_hbm.at[pl.ds(base0 + c * CB, CB)],
                              dstc.at[k], semd[k]).wait()

        @pl.loop(0, CB, step=16)
        def _(i):
            p16 = pk_v[pl.ds(c * CB + i, 16)]
            s16 = sc[k][pl.ds(i, 16)]
            d16 = dstc[k, pl.ds(i, 16)]
            x = (plsc.load_gather(ft_v, [s16])
                 + plsc.load_gather(ft_v, [d16])
                 + plsc.load_gather(ee_v, [p16 >> 14]))
            a_v[pl.ds(i, 16)] = jnp.maximum(x, x * 0.02)

        pltpu.make_async_copy(h_hbm.at[sc[k]], rows[k], semg[k]).wait()

        @pl.loop(0, CB)
        def _(e):
            ak = plsc.load_gather(a_v, [jnp.full((16,), e, jnp.int32)])
            for j in range(D // 16):
                sl = pl.ds(j * 16, 16)
                rows[k][e, sl] = rows[k][e, sl] * ak

        pltpu.sync_copy(rows[k], agg_sh.at[dstc.at[k]], add=True)

    prefetch(0, 0)

    @pl.loop(0, NCHUNK + 1, step=2)
    def _(c):
        for k in range(2):
            t = c + k

            @pl.when(t < NCHUNK)
            def _():
                @pl.when(t + 1 < NCHUNK)
                def _():
                    prefetch((k + 1) % 2, t + 1)

                process(k, t)

    plsc.subcore_barrier()

    @pl.loop(0, ZCH)
    def _(k):
        @pl.when(lax.rem(k, NS) == sid)
        def _():
            pltpu.sync_copy(agg_sh.at[pl.ds(k * CB, CB)],
                            out_hbm.at[cid, pl.ds(k * CB, CB)])


def _post_body(aggp_ref, w_ref, b_ref, gamma_ref, beta_ref, out_ref):
    agg = aggp_ref[0] + aggp_ref[1]
    y = lax.dot_general(agg, w_ref[...], (((1,), (1,)), ((), ())),
                        preferred_element_type=jnp.float32)
    y = y + b_ref[...]
    mean = jnp.mean(y, axis=0, keepdims=True)
    var = jnp.mean((y - mean) ** 2, axis=0, keepdims=True)
    yn = (y - mean) / jnp.sqrt(var + 1e-5) * gamma_ref[...] + beta_ref[...]
    out_ref[...] = jnp.where(yn > 0, yn, jnp.exp(jnp.minimum(yn, 0.0)) - 1.0)


def kernel(nodes_feat, edge_index, edges_feat, nodes_fc, edges_fc, edge_emb,
           nodes_attn, edges_attn, W, b, gamma, beta):
    h, ft2, ee2 = pl.pallas_call(
        _pre_body,
        out_shape=(
            jax.ShapeDtypeStruct((N, D), jnp.float32),
            jax.ShapeDtypeStruct((N, 1), jnp.float32),
            jax.ShapeDtypeStruct((16, 1), jnp.float32),
        ),
    )(nodes_feat, nodes_fc, nodes_attn, edge_emb, edges_fc, edges_attn)
    ft = ft2.reshape(N)
    ee = ee2.reshape(16)

    pk = edge_index[0] | (edges_feat << 14)
    dst = edge_index[1]

    cp = pltpu.CompilerParams()
    if "needs_layout_passes" in pltpu.CompilerParams.__dataclass_fields__:
        cp = dataclasses.replace(cp, needs_layout_passes=False)
    mesh = plsc.VectorSubcoreMesh(core_axis_name="c", subcore_axis_name="s")
    edge_kernel = pl.kernel(
        _edge_body,
        compiler_params=cp,
        out_type=jax.ShapeDtypeStruct((NC, N, D), jnp.float32),
        mesh=mesh,
        scratch_types=[
            pltpu.VMEM((N,), jnp.float32),
            pltpu.VMEM((16,), jnp.float32),
            pltpu.VMEM((EPW,), jnp.int32),
            pltpu.VMEM((2, CB), jnp.int32),
            pltpu.VMEM((CB,), jnp.float32),
            pltpu.VMEM((CB,), jnp.int32),
            pltpu.VMEM((CB,), jnp.int32),
            pltpu.VMEM((CB, D), jnp.float32),
            pltpu.VMEM((CB, D), jnp.float32),
            pltpu.VMEM_SHARED((N, D), jnp.float32),
            pltpu.SemaphoreType.DMA,
            pltpu.SemaphoreType.DMA,
            pltpu.SemaphoreType.DMA,
            pltpu.SemaphoreType.DMA,
        ],
    )
    aggp = edge_kernel(h, ft, ee, pk, dst)

    out = pl.pallas_call(
        _post_body,
        out_shape=jax.ShapeDtypeStruct((N, D), jnp.float32),
    )(aggp, W, b.reshape(1, D), gamma.reshape(1, D), beta.reshape(1, D))
    return out


# speedup vs baseline: 30.5904x; 1.1260x over previous
import dataclasses
import functools

import jax
import jax.numpy as jnp
from jax import lax
from jax.experimental import pallas as pl
from jax.experimental.pallas import tpu as pltpu
from jax.experimental.pallas import tpu_sc as plsc

N = 10000
E = 320000
D = 128

NC = 2
NS = 16
NW = NC * NS
EPW = E // NW
CB = 80
NCHUNK = EPW // CB
ZCH = N // CB


def _pre_body(nf_ref, nfc_ref, nattn_ref, eemb_ref, efc_ref, eattn_ref,
              h_ref, ft_ref, ee_ref):
    hv = nf_ref[...] * nfc_ref[0:1, :]
    h_ref[...] = hv
    ft_ref[...] = jnp.sum(hv * nattn_ref[...], axis=1, keepdims=True)
    ee = jnp.sum(eemb_ref[...] * efc_ref[...] * eattn_ref[...], axis=1,
                 keepdims=True)
    ee_ref[...] = jnp.concatenate([ee, jnp.zeros((8, 1), jnp.float32)], axis=0)


def _edge_body(h_hbm, ft_hbm, ee_hbm, pk_hbm, out_hbm,
               ee_v, pk_v, a_v, sc, dstc, fts, ftd, rows, agg_sh,
               semg, semf, semh, sems):
    cid = lax.axis_index("c")
    sid = lax.axis_index("s")
    wid = cid * NS + sid
    base0 = wid * EPW

    pltpu.sync_copy(ee_hbm, ee_v)
    pltpu.sync_copy(pk_hbm.at[pl.ds(base0, EPW)], pk_v)

    @pl.loop(0, CB)
    def _(r):
        for j in range(D // 16):
            rows[0, r, pl.ds(j * 16, 16)] = jnp.zeros((16,), jnp.float32)

    @pl.loop(0, ZCH)
    def _(k):
        @pl.when(lax.rem(k, NS) == sid)
        def _():
            pltpu.sync_copy(rows.at[0], agg_sh.at[pl.ds(k * CB, CB)])

    plsc.subcore_barrier()

    def prefetch(k, c):
        @pl.loop(0, CB, step=16)
        def _(i):
            p16 = pk_v[pl.ds(c * CB + i, 16)]
            sc[k, pl.ds(i, 16)] = p16 & 0x3FFF
            dstc[k, pl.ds(i, 16)] = lax.shift_right_logical(p16, 17)

        pltpu.async_copy(h_hbm.at[sc.at[k]], rows.at[k], semg.at[k])
        pltpu.async_copy(ft_hbm.at[sc.at[k]], fts.at[k], semf.at[k])
        pltpu.async_copy(ft_hbm.at[dstc.at[k]], ftd.at[k], semh.at[k])

    def process(k, c):
        pltpu.make_async_copy(ft_hbm.at[sc.at[k]], fts.at[k],
                              semf.at[k]).wait()
        pltpu.make_async_copy(ft_hbm.at[dstc.at[k]], ftd.at[k],
                              semh.at[k]).wait()

        @pl.loop(0, CB, step=16)
        def _(i):
            et16 = lax.shift_right_logical(pk_v[pl.ds(c * CB + i, 16)], 14) & 7
            x = (fts[k, pl.ds(i, 16)] + ftd[k, pl.ds(i, 16)]
                 + plsc.load_gather(ee_v, [et16]))
            a_v[pl.ds(i, 16)] = jnp.maximum(x, x * 0.02)

        pltpu.make_async_copy(h_hbm.at[sc.at[k]], rows.at[k],
                              semg.at[k]).wait()

        @pl.loop(0, CB)
        def _(e):
            ak = plsc.load_gather(a_v, [jnp.full((16,), e, jnp.int32)])
            for j in range(D // 16):
                sl = pl.ds(j * 16, 16)
                rows[k, e, sl] = rows[k, e, sl] * ak

        pltpu.async_copy(rows.at[k], agg_sh.at[dstc.at[k]], sems.at[k],
                         add=True)

    def drain_scatter(k):
        pltpu.make_async_copy(rows.at[k], agg_sh.at[dstc.at[k]],
                              sems.at[k]).wait()

    prefetch(0, 0)
    prefetch(1, 1)

    @pl.loop(0, NCHUNK + 1, step=3)
    def _(c):
        for k3 in range(3):
            t = c + k3

            @pl.when(t < NCHUNK)
            def _():
                process(k3, t)

                @pl.when(t + 2 < NCHUNK)
                def _():
                    @pl.when(t >= 1)
                    def _():
                        drain_scatter((k3 + 2) % 3)

                    prefetch((k3 + 2) % 3, t + 2)

    for k3 in range(3):
        drain_scatter(k3)

    plsc.subcore_barrier()

    @pl.loop(0, ZCH)
    def _(k):
        @pl.when(lax.rem(k, NS) == sid)
        def _():
            pltpu.sync_copy(agg_sh.at[pl.ds(k * CB, CB)],
                            out_hbm.at[cid, pl.ds(k * CB, CB)])


def _post_body(aggp_ref, w_ref, b_ref, gamma_ref, beta_ref, out_ref):
    agg = aggp_ref[0] + aggp_ref[1]
    y = lax.dot_general(agg, w_ref[...], (((1,), (1,)), ((), ())),
                        preferred_element_type=jnp.float32)
    y = y + b_ref[...]
    mean = jnp.mean(y, axis=0, keepdims=True)
    var = jnp.mean((y - mean) ** 2, axis=0, keepdims=True)
    yn = (y - mean) / jnp.sqrt(var + 1e-5) * gamma_ref[...] + beta_ref[...]
    out_ref[...] = jnp.where(yn > 0, yn, jnp.exp(jnp.minimum(yn, 0.0)) - 1.0)


def kernel(nodes_feat, edge_index, edges_feat, nodes_fc, edges_fc, edge_emb,
           nodes_attn, edges_attn, W, b, gamma, beta):
    h, ft2, ee2 = pl.pallas_call(
        _pre_body,
        out_shape=(
            jax.ShapeDtypeStruct((N, D), jnp.float32),
            jax.ShapeDtypeStruct((N, 1), jnp.float32),
            jax.ShapeDtypeStruct((16, 1), jnp.float32),
        ),
    )(nodes_feat, nodes_fc, nodes_attn, edge_emb, edges_fc, edges_attn)
    ft = ft2.reshape(N)
    ee = ee2.reshape(16)

    pk = edge_index[0] | (edges_feat << 14) | (edge_index[1] << 17)

    cp = pltpu.CompilerParams()
    if "needs_layout_passes" in pltpu.CompilerParams.__dataclass_fields__:
        cp = dataclasses.replace(cp, needs_layout_passes=False)
    mesh = plsc.VectorSubcoreMesh(core_axis_name="c", subcore_axis_name="s")
    edge_kernel = pl.kernel(
        _edge_body,
        compiler_params=cp,
        out_type=jax.ShapeDtypeStruct((NC, N, D), jnp.float32),
        mesh=mesh,
        scratch_types=[
            pltpu.VMEM((16,), jnp.float32),
            pltpu.VMEM((EPW,), jnp.int32),
            pltpu.VMEM((CB,), jnp.float32),
            pltpu.VMEM((3, CB), jnp.int32),
            pltpu.VMEM((3, CB), jnp.int32),
            pltpu.VMEM((3, CB), jnp.float32),
            pltpu.VMEM((3, CB), jnp.float32),
            pltpu.VMEM((3, CB, D), jnp.float32),
            pltpu.VMEM_SHARED((N, D), jnp.float32),
            pltpu.SemaphoreType.DMA((3,)),
            pltpu.SemaphoreType.DMA((3,)),
            pltpu.SemaphoreType.DMA((3,)),
            pltpu.SemaphoreType.DMA((3,)),
        ],
    )
    aggp = edge_kernel(h, ft, ee, pk)

    out = pl.pallas_call(
        _post_body,
        out_shape=jax.ShapeDtypeStruct((N, D), jnp.float32),
    )(aggp, W, b.reshape(1, D), gamma.reshape(1, D), beta.reshape(1, D))
    return out
